# baseline (speedup 1.0000x reference)
import functools

import jax
import jax.numpy as jnp
from jax import lax
from jax.experimental import pallas as pl
from jax.experimental.pallas import tpu as pltpu
from jax.experimental.pallas import tpu_sc as plsc

_N = 10000
_E = 320000
_D = 128
_NPAD = 10240
_NC, _NS = 2, 16
_CH = 128
_RPT = _NPAD // _NS
_ZR = 80
_ZCH = _RPT // _ZR
_BR = 1024


def _make_sc_scatter(d, nchunk):
    mesh = plsc.VectorSubcoreMesh(core_axis_name="c", subcore_axis_name="s")
    out_t = [jax.ShapeDtypeStruct((_NPAD, d), jnp.float32),
             jax.ShapeDtypeStruct((_NPAD, d), jnp.float32)]

    assert nchunk % 2 == 1

    @functools.partial(
        pl.kernel, out_type=out_t, mesh=mesh,
        compiler_params=pltpu.CompilerParams(
            use_tc_tiling_on_sc=(d == _D),
            disable_bounds_checks=True,
            disable_semaphore_checks=True,
        ),
        scratch_types=[
            pltpu.VMEM((1, 2, _CH), jnp.int32),
            pltpu.VMEM((1, 2, _CH), jnp.int32),
            pltpu.VMEM((_CH, d), jnp.float32),
            pltpu.VMEM((_CH, d), jnp.float32),
            pltpu.VMEM_SHARED((_NPAD, d), jnp.float32),
            pltpu.SemaphoreType.DMA,
            pltpu.SemaphoreType.DMA,
        ],
    )
    def k(sd_hbm, tbl_hbm, zrow_hbm, out0, out1,
          sd0, sd1, rows0, rows1, acc, sem0, sem1):
        c = lax.axis_index("c")
        s = lax.axis_index("s")
        wid = c * _NS + s
        base = wid * nchunk
        pltpu.sync_copy(zrow_hbm, rows0)
        for kk in range(_ZCH):
            pltpu.sync_copy(rows0.at[pl.ds(0, _ZR)],
                            acc.at[pl.ds(s * _RPT + kk * _ZR, _ZR)])
        plsc.subcore_barrier()

        pltpu.sync_copy(sd_hbm.at[pl.ds(base, 1)], sd0)
        pltpu.async_copy(tbl_hbm.at[sd0.at[0, 0]], rows0, sem0)
        pltpu.sync_copy(sd_hbm.at[pl.ds(base + 1, 1)], sd1)

        def body(jj, carry):
            j0 = 2 * jj
            pltpu.make_async_copy(tbl_hbm.at[sd0.at[0, 0]], rows0,
                                  sem0).wait()
            pltpu.async_copy(tbl_hbm.at[sd1.at[0, 0]], rows1, sem1)
            pltpu.sync_copy(rows0, acc.at[sd0.at[0, 1]], add=True)
            pltpu.sync_copy(sd_hbm.at[pl.ds(base + j0 + 2, 1)], sd0)
            pltpu.make_async_copy(tbl_hbm.at[sd1.at[0, 0]], rows1,
                                  sem1).wait()
            pltpu.async_copy(tbl_hbm.at[sd0.at[0, 0]], rows0, sem0)
            pltpu.sync_copy(rows1, acc.at[sd1.at[0, 1]], add=True)
            pltpu.sync_copy(sd_hbm.at[pl.ds(base + j0 + 3, 1)], sd1)
            return carry

        lax.fori_loop(0, (nchunk - 1) // 2, body, 0)
        pltpu.make_async_copy(tbl_hbm.at[sd0.at[0, 0]], rows0, sem0).wait()
        pltpu.sync_copy(rows0, acc.at[sd0.at[0, 1]], add=True)
        plsc.subcore_barrier()
        for kk in range(_ZCH):
            r0 = s * _RPT + kk * _ZR
            pltpu.sync_copy(acc.at[pl.ds(r0, _ZR)], rows0.at[pl.ds(0, _ZR)])

            @pl.when(c == 0)
            def _():
                pltpu.sync_copy(rows0.at[pl.ds(0, _ZR)],
                                out0.at[pl.ds(r0, _ZR)])

            @pl.when(c == 1)
            def _():
                pltpu.sync_copy(rows0.at[pl.ds(0, _ZR)],
                                out1.at[pl.ds(r0, _ZR)])

    return k


def _row_spec(d):
    return pl.BlockSpec((_BR, d), lambda i: (i, 0))


def _full_spec(shape):
    return pl.BlockSpec(shape, lambda i: (0,) * len(shape))


def _tc_stage(t=None, p=None, dinv16=None, bias=None, res=None, relu=False,
              W=None, extra=None, scale=True, emit_t=False):
    args, specs, layout = [], [], []

    def add_rows(a, name):
        args.append(a)
        specs.append(_row_spec(a.shape[1]))
        layout.append(name)

    if p is not None:
        add_rows(p[0], "p0")
        add_rows(p[1], "p1")
        add_rows(dinv16, "dinv")
        b2 = bias.reshape(1, -1)
        args.append(b2)
        specs.append(_full_spec(b2.shape))
        layout.append("bias")
        if res is not None:
            add_rows(res, "res")
        d_in = p[0].shape[1]
    else:
        add_rows(t, "t")
        if W is not None and scale:
            add_rows(dinv16, "dinv")
        d_in = t.shape[1]
    if W is not None:
        args.append(W)
        specs.append(_full_spec(W.shape))
        layout.append("W")
        if extra is not None:
            add_rows(extra, "extra")

    outs, out_specs = [], []
    if W is not None:
        outs.append(jax.ShapeDtypeStruct((_NPAD, W.shape[1]), jnp.float32))
        out_specs.append(_row_spec(W.shape[1]))
    if emit_t:
        outs.append(jax.ShapeDtypeStruct((_NPAD, d_in), jnp.float32))
        out_specs.append(_row_spec(d_in))

    has_p = p is not None
    has_res = res is not None
    has_extra = extra is not None
    has_W = W is not None

    def body(*refs):
        vals = {name: r[...] for name, r in zip(layout, refs[:len(layout)])}
        orefs = refs[len(layout):]
        if has_p:
            dv = vals["dinv"][:, :1]
            tt = dv * (vals["p0"] + vals["p1"]) + vals["bias"]
            if has_res:
                tt = tt + vals["res"]
            if relu:
                tt = jnp.maximum(tt, 0.0)
        else:
            tt = vals["t"]
        oi = 0
        if has_W:
            g = lax.dot_general(tt, vals["W"], (((1,), (0,)), ((), ())),
                                preferred_element_type=jnp.float32)
            if has_extra:
                g = g + vals["extra"]
            if scale:
                g = g * vals["dinv"][:, :1]
            orefs[oi][...] = g
            oi += 1
        if emit_t:
            orefs[oi][...] = tt

    r = pl.pallas_call(body, grid=(_NPAD // _BR,), in_specs=specs,
                       out_specs=out_specs, out_shape=outs)(*args)
    return r[0] if len(outs) == 1 else r


def _prep_inputs(src2, dst2):
    nrow = src2.shape[0]

    def body(s_ref, d_ref, so, do, ones_o, z128_o, z32_o, z16_o):
        so[...] = s_ref[...]
        do[...] = d_ref[...]
        ones_o[...] = jnp.ones_like(ones_o)
        z128_o[...] = jnp.zeros_like(z128_o)
        z32_o[...] = jnp.zeros_like(z32_o)
        z16_o[...] = jnp.zeros_like(z16_o)

    return pl.pallas_call(
        body,
        out_shape=[
            jax.ShapeDtypeStruct((nrow, _CH), jnp.int32),
            jax.ShapeDtypeStruct((nrow, _CH), jnp.int32),
            jax.ShapeDtypeStruct((_NPAD, 16), jnp.float32),
            jax.ShapeDtypeStruct((_CH, _D), jnp.float32),
            jax.ShapeDtypeStruct((_CH, 32), jnp.float32),
            jax.ShapeDtypeStruct((_CH, 16), jnp.float32),
        ])(src2, dst2)


def _dinv_from_deg(dp0, dp1):
    def body(a, b, o):
        o[...] = lax.rsqrt(jnp.maximum(a[...] + b[...], 1.0))

    return pl.pallas_call(
        body, grid=(_NPAD // _BR,),
        in_specs=[_row_spec(16), _row_spec(16)],
        out_specs=_row_spec(16),
        out_shape=jax.ShapeDtypeStruct((_NPAD, 16), jnp.float32))(dp0, dp1)


def kernel(x, edge_index, iters_to_do, Wp, bp, Wr, br, W1a, b1a, W1b, b1b,
           W2a, b2a, W2b, b2b, Wh1, bh1, Wh2, bh2, Wh3, bh3):
    n, e = _N, _E
    e2 = e + n
    nchunk = -(-e2 // (_NC * _NS * _CH))
    if nchunk % 2 == 0:
        nchunk += 1
    e2p = nchunk * _NC * _NS * _CH

    loops = jnp.arange(n, dtype=jnp.int32)
    src = jnp.concatenate([edge_index[0], loops,
                           jnp.zeros((e2p - e2,), jnp.int32)])
    dst = jnp.concatenate([edge_index[1], loops,
                           jnp.full((e2p - e2,), n, jnp.int32)])
    sd = jnp.stack([src.reshape(-1, _CH), dst.reshape(-1, _CH)], axis=1)
    sd = jnp.pad(sd, ((0, 1), (0, 0), (0, 0)))

    xp = jnp.zeros((_NPAD, _D), jnp.float32).at[:n].set(x)
    ones16 = jnp.ones((_NPAD, 16), jnp.float32)
    z128 = jnp.zeros((_CH, _D), jnp.float32)
    z32 = jnp.zeros((_CH, 32), jnp.float32)
    z16 = jnp.zeros((_CH, 16), jnp.float32)

    sc128 = _make_sc_scatter(_D, nchunk)
    sc32 = _make_sc_scatter(32, nchunk)
    sc16 = _make_sc_scatter(16, nchunk)

    def scat128(tbl):
        return sc128(sd, tbl, z128)

    Wr_top, Wr_bot = Wr[:_D], Wr[_D:]
    Wh2p = jnp.zeros((32, 16), jnp.float32).at[:, :8].set(Wh2)
    bh2p = jnp.zeros((16,), jnp.float32).at[:8].set(bh2)
    Wh3p = jnp.zeros((16, 16), jnp.float32).at[:8, :2].set(Wh3)
    bh3p = jnp.zeros((16,), jnp.float32).at[:2].set(bh3)

    dp0, dp1 = sc16(sd, ones16, z16)
    dinv16 = _dinv_from_deg(dp0, dp1)

    xr = _tc_stage(t=xp, W=Wr_bot, scale=False)
    Gp = _tc_stage(t=xp, W=Wp, dinv16=dinv16)

    pr = scat128(Gp)
    G, interim = _tc_stage(p=pr, dinv16=dinv16, bias=bp, relu=True,
                           W=Wr_top, extra=xr, emit_t=True)

    def body(_, carry):
        G, interim = carry
        pq = scat128(G)
        G1, h = _tc_stage(p=pq, dinv16=dinv16, bias=br, relu=False,
                          W=W1a, emit_t=True)
        pq = scat128(G1)
        G2 = _tc_stage(p=pq, dinv16=dinv16, bias=b1a, relu=True, W=W1b)
        pq = scat128(G2)
        G3, h2 = _tc_stage(p=pq, dinv16=dinv16, bias=b1b, relu=True, res=h,
                           W=W2a, emit_t=True)
        pq = scat128(G3)
        G4 = _tc_stage(p=pq, dinv16=dinv16, bias=b2a, relu=True, W=W2b)
        pq = scat128(G4)
        G5, interim2 = _tc_stage(p=pq, dinv16=dinv16, bias=b2b, relu=True,
                                 res=h2, W=Wr_top, extra=xr, emit_t=True)
        return (G5, interim2)

    G, interim = lax.fori_loop(0, iters_to_do, body, (G, interim))

    Gh = _tc_stage(t=interim, W=Wh1, dinv16=dinv16)
    ph = sc32(sd, Gh, z32)
    Gh = _tc_stage(p=ph, dinv16=dinv16, bias=bh1, relu=True, W=Wh2p)
    ph = sc16(sd, Gh, z16)
    Gh = _tc_stage(p=ph, dinv16=dinv16, bias=bh2p, relu=True, W=Wh3p)
    ph = sc16(sd, Gh, z16)
    out16 = _tc_stage(p=ph, dinv16=dinv16, bias=bh3p, relu=False, emit_t=True)

    out = out16[:n, :2]
    return jnp.where(iters_to_do > 0, out, jnp.zeros_like(out))


# speedup vs baseline: 1.0019x; 1.0019x over previous
import functools

import jax
import jax.numpy as jnp
from jax import lax
from jax.experimental import pallas as pl
from jax.experimental.pallas import tpu as pltpu
from jax.experimental.pallas import tpu_sc as plsc

_N = 10000
_E = 320000
_D = 128
_NPAD = 10240
_NC, _NS = 2, 16
_CH = 128
_NACC = 10112
_RPT = _NACC // _NS
_ZRS = (80, 80, 80, 80, 80, 80, 80, 72)
_BR = 1024


def _make_sc_scatter(d, nchunk):
    mesh = plsc.VectorSubcoreMesh(core_axis_name="c", subcore_axis_name="s")
    out_t = [jax.ShapeDtypeStruct((_NPAD, d), jnp.float32),
             jax.ShapeDtypeStruct((_NPAD, d), jnp.float32)]

    assert nchunk % 3 == 0

    @functools.partial(
        pl.kernel, out_type=out_t, mesh=mesh,
        compiler_params=pltpu.CompilerParams(
            use_tc_tiling_on_sc=(d == _D),
            disable_bounds_checks=True,
            disable_semaphore_checks=True,
        ),
        scratch_types=[
            [pltpu.VMEM((1, 2, _CH), jnp.int32)] * 3,
            [pltpu.VMEM((_CH, d), jnp.float32)] * 3,
            pltpu.VMEM_SHARED((_NACC, d), jnp.float32),
            [pltpu.SemaphoreType.DMA] * 3,
            [pltpu.SemaphoreType.DMA] * 3,
        ],
    )
    def k(sd_hbm, tbl_hbm, zrow_hbm, out0, out1,
          sd, rows, acc, gsem, ssem):
        c = lax.axis_index("c")
        s = lax.axis_index("s")
        wid = c * _NS + s
        base = wid * nchunk
        pltpu.sync_copy(zrow_hbm, rows[0])
        r0 = s * _RPT
        for zr in _ZRS:
            pltpu.sync_copy(rows[0].at[pl.ds(0, zr)], acc.at[pl.ds(r0, zr)])
            r0 += zr
        plsc.subcore_barrier()

        pltpu.sync_copy(sd_hbm.at[pl.ds(base, 1)], sd[0])
        pltpu.async_copy(tbl_hbm.at[sd[0].at[0, 0]], rows[0], gsem[0])

        def body(jj, carry):
            j0 = 3 * jj
            for db in range(3):
                j = j0 + db
                b = db
                b1 = (db + 1) % 3

                @pl.when(j >= 2)
                def _():
                    pltpu.make_async_copy(rows[b1], acc.at[sd[b1].at[0, 1]],
                                          ssem[b1]).wait()

                @pl.when(j + 1 < nchunk)
                def _():
                    pltpu.sync_copy(sd_hbm.at[pl.ds(base + j + 1, 1)],
                                    sd[b1])
                pltpu.make_async_copy(tbl_hbm.at[sd[b].at[0, 0]], rows[b],
                                      gsem[b]).wait()

                @pl.when(j + 1 < nchunk)
                def _():
                    pltpu.async_copy(tbl_hbm.at[sd[b1].at[0, 0]], rows[b1],
                                     gsem[b1])
                pltpu.async_copy(rows[b], acc.at[sd[b].at[0, 1]], ssem[b],
                                 add=True)
            return carry

        lax.fori_loop(0, nchunk // 3, body, 0)
        for b in ((nchunk - 2) % 3, (nchunk - 1) % 3):
            pltpu.make_async_copy(rows[b], acc.at[sd[b].at[0, 1]],
                                  ssem[b]).wait()
        plsc.subcore_barrier()
        r0 = s * _RPT
        for zr in _ZRS:
            pltpu.sync_copy(acc.at[pl.ds(r0, zr)], rows[0].at[pl.ds(0, zr)])

            @pl.when(c == 0)
            def _():
                pltpu.sync_copy(rows[0].at[pl.ds(0, zr)],
                                out0.at[pl.ds(r0, zr)])

            @pl.when(c == 1)
            def _():
                pltpu.sync_copy(rows[0].at[pl.ds(0, zr)],
                                out1.at[pl.ds(r0, zr)])
            r0 += zr

    return k


def _row_spec(d):
    return pl.BlockSpec((_BR, d), lambda i: (i, 0))


def _full_spec(shape):
    return pl.BlockSpec(shape, lambda i: (0,) * len(shape))


def _tc_stage(t=None, p=None, dinv16=None, bias=None, res=None, relu=False,
              W=None, extra=None, scale=True, emit_t=False):
    args, specs, layout = [], [], []

    def add_rows(a, name):
        args.append(a)
        specs.append(_row_spec(a.shape[1]))
        layout.append(name)

    if p is not None:
        add_rows(p[0], "p0")
        add_rows(p[1], "p1")
        add_rows(dinv16, "dinv")
        b2 = bias.reshape(1, -1)
        args.append(b2)
        specs.append(_full_spec(b2.shape))
        layout.append("bias")
        if res is not None:
            add_rows(res, "res")
        d_in = p[0].shape[1]
    else:
        add_rows(t, "t")
        if W is not None and scale:
            add_rows(dinv16, "dinv")
        d_in = t.shape[1]
    if W is not None:
        args.append(W)
        specs.append(_full_spec(W.shape))
        layout.append("W")
        if extra is not None:
            add_rows(extra, "extra")

    outs, out_specs = [], []
    if W is not None:
        outs.append(jax.ShapeDtypeStruct((_NPAD, W.shape[1]), jnp.float32))
        out_specs.append(_row_spec(W.shape[1]))
    if emit_t:
        outs.append(jax.ShapeDtypeStruct((_NPAD, d_in), jnp.float32))
        out_specs.append(_row_spec(d_in))

    has_p = p is not None
    has_res = res is not None
    has_extra = extra is not None
    has_W = W is not None

    def body(*refs):
        vals = {name: r[...] for name, r in zip(layout, refs[:len(layout)])}
        orefs = refs[len(layout):]
        if has_p:
            dv = vals["dinv"][:, :1]
            tt = dv * (vals["p0"] + vals["p1"]) + vals["bias"]
            if has_res:
                tt = tt + vals["res"]
            if relu:
                tt = jnp.maximum(tt, 0.0)
        else:
            tt = vals["t"]
        oi = 0
        if has_W:
            g = lax.dot_general(tt, vals["W"], (((1,), (0,)), ((), ())),
                                preferred_element_type=jnp.float32)
            if has_extra:
                g = g + vals["extra"]
            if scale:
                g = g * vals["dinv"][:, :1]
            orefs[oi][...] = g
            oi += 1
        if emit_t:
            orefs[oi][...] = tt

    r = pl.pallas_call(body, grid=(_NPAD // _BR,), in_specs=specs,
                       out_specs=out_specs, out_shape=outs)(*args)
    return r[0] if len(outs) == 1 else r


def _prep_inputs(src2, dst2):
    nrow = src2.shape[0]

    def body(s_ref, d_ref, so, do, ones_o, z128_o, z32_o, z16_o):
        so[...] = s_ref[...]
        do[...] = d_ref[...]
        ones_o[...] = jnp.ones_like(ones_o)
        z128_o[...] = jnp.zeros_like(z128_o)
        z32_o[...] = jnp.zeros_like(z32_o)
        z16_o[...] = jnp.zeros_like(z16_o)

    return pl.pallas_call(
        body,
        out_shape=[
            jax.ShapeDtypeStruct((nrow, _CH), jnp.int32),
            jax.ShapeDtypeStruct((nrow, _CH), jnp.int32),
            jax.ShapeDtypeStruct((_NPAD, 16), jnp.float32),
            jax.ShapeDtypeStruct((_CH, _D), jnp.float32),
            jax.ShapeDtypeStruct((_CH, 32), jnp.float32),
            jax.ShapeDtypeStruct((_CH, 16), jnp.float32),
        ])(src2, dst2)


def _dinv_from_deg(dp0, dp1):
    def body(a, b, o):
        o[...] = lax.rsqrt(jnp.maximum(a[...] + b[...], 1.0))

    return pl.pallas_call(
        body, grid=(_NPAD // _BR,),
        in_specs=[_row_spec(16), _row_spec(16)],
        out_specs=_row_spec(16),
        out_shape=jax.ShapeDtypeStruct((_NPAD, 16), jnp.float32))(dp0, dp1)


def kernel(x, edge_index, iters_to_do, Wp, bp, Wr, br, W1a, b1a, W1b, b1b,
           W2a, b2a, W2b, b2b, Wh1, bh1, Wh2, bh2, Wh3, bh3):
    n, e = _N, _E
    e2 = e + n
    nchunk = -(-e2 // (_NC * _NS * _CH))
    nchunk += -nchunk % 3
    e2p = nchunk * _NC * _NS * _CH

    loops = jnp.arange(n, dtype=jnp.int32)
    src = jnp.concatenate([edge_index[0], loops,
                           jnp.zeros((e2p - e2,), jnp.int32)])
    dst = jnp.concatenate([edge_index[1], loops,
                           jnp.full((e2p - e2,), n, jnp.int32)])
    sd = jnp.stack([src.reshape(-1, _CH), dst.reshape(-1, _CH)], axis=1)
    sd = jnp.pad(sd, ((0, 1), (0, 0), (0, 0)))

    xp = jnp.zeros((_NPAD, _D), jnp.float32).at[:n].set(x)
    ones16 = jnp.ones((_NPAD, 16), jnp.float32)
    z128 = jnp.zeros((_CH, _D), jnp.float32)
    z32 = jnp.zeros((_CH, 32), jnp.float32)
    z16 = jnp.zeros((_CH, 16), jnp.float32)

    sc128 = _make_sc_scatter(_D, nchunk)
    sc32 = _make_sc_scatter(32, nchunk)
    sc16 = _make_sc_scatter(16, nchunk)

    def scat128(tbl):
        return sc128(sd, tbl, z128)

    Wr_top, Wr_bot = Wr[:_D], Wr[_D:]
    Wh2p = jnp.zeros((32, 16), jnp.float32).at[:, :8].set(Wh2)
    bh2p = jnp.zeros((16,), jnp.float32).at[:8].set(bh2)
    Wh3p = jnp.zeros((16, 16), jnp.float32).at[:8, :2].set(Wh3)
    bh3p = jnp.zeros((16,), jnp.float32).at[:2].set(bh3)

    dp0, dp1 = sc16(sd, ones16, z16)
    dinv16 = _dinv_from_deg(dp0, dp1)

    xr = _tc_stage(t=xp, W=Wr_bot, scale=False)
    Gp = _tc_stage(t=xp, W=Wp, dinv16=dinv16)

    pr = scat128(Gp)
    G, interim = _tc_stage(p=pr, dinv16=dinv16, bias=bp, relu=True,
                           W=Wr_top, extra=xr, emit_t=True)

    def body(_, carry):
        G, interim = carry
        pq = scat128(G)
        G1, h = _tc_stage(p=pq, dinv16=dinv16, bias=br, relu=False,
                          W=W1a, emit_t=True)
        pq = scat128(G1)
        G2 = _tc_stage(p=pq, dinv16=dinv16, bias=b1a, relu=True, W=W1b)
        pq = scat128(G2)
        G3, h2 = _tc_stage(p=pq, dinv16=dinv16, bias=b1b, relu=True, res=h,
                           W=W2a, emit_t=True)
        pq = scat128(G3)
        G4 = _tc_stage(p=pq, dinv16=dinv16, bias=b2a, relu=True, W=W2b)
        pq = scat128(G4)
        G5, interim2 = _tc_stage(p=pq, dinv16=dinv16, bias=b2b, relu=True,
                                 res=h2, W=Wr_top, extra=xr, emit_t=True)
        return (G5, interim2)

    G, interim = lax.fori_loop(0, iters_to_do, body, (G, interim))

    Gh = _tc_stage(t=interim, W=Wh1, dinv16=dinv16)
    ph = sc32(sd, Gh, z32)
    Gh = _tc_stage(p=ph, dinv16=dinv16, bias=bh1, relu=True, W=Wh2p)
    ph = sc16(sd, Gh, z16)
    Gh = _tc_stage(p=ph, dinv16=dinv16, bias=bh2p, relu=True, W=Wh3p)
    ph = sc16(sd, Gh, z16)
    out16 = _tc_stage(p=ph, dinv16=dinv16, bias=bh3p, relu=False, emit_t=True)

    out = out16[:n, :2]
    return jnp.where(iters_to_do > 0, out, jnp.zeros_like(out))
